# baseline (speedup 1.0000x reference)
import functools
import math

import jax
import jax.numpy as jnp
from jax import lax
from jax.experimental import pallas as pl
from jax.experimental.pallas import tpu as pltpu
from jax.experimental.pallas import tpu_sc as plsc

VOCAB_SIZE = 1000000
D = 64
SCALE = math.sqrt(D)
NC, NS = 2, 16
NW = NC * NS
B = 4096 * 200
K = 128
ROWS_PER_W = B // NW
CH = ROWS_PER_W // K
LANES = 16


def _scale_buf(buf):
    @pl.loop(0, K)
    def _row(r):
        for c in range(D // LANES):
            sl = pl.ds(c * LANES, LANES)
            buf[r, sl] = buf[r, sl] * SCALE


def _make_sc_kernel():
    mesh = plsc.VectorSubcoreMesh(core_axis_name="c", subcore_axis_name="s")

    @functools.partial(
        pl.kernel,
        out_type=jax.ShapeDtypeStruct((B, D), jnp.float32),
        mesh=mesh,
        compiler_params=pltpu.CompilerParams(use_tc_tiling_on_sc=False),
        scratch_types=[
            pltpu.VMEM((CH, K), jnp.int32),
            pltpu.VMEM((K, D), jnp.float32),
            pltpu.VMEM((K, D), jnp.float32),
            pltpu.SemaphoreType.DMA,
            pltpu.SemaphoreType.DMA,
        ],
    )
    def emb(x_hbm, tab_hbm, out_hbm, idx_v, buf0, buf1, sem0, sem1):
        wid = lax.axis_index("s") * NC + lax.axis_index("c")
        pltpu.sync_copy(x_hbm.at[pl.ds(wid * CH, CH)], idx_v)
        base = wid * ROWS_PER_W

        bufs = (buf0, buf1)
        sems = (sem0, sem1)
        for b in range(2):
            pltpu.async_copy(tab_hbm.at[idx_v.at[b]], bufs[b], sems[b])

        @pl.loop(0, CH, step=2)
        def _chunk(j):
            for b in range(2):
                jj = j + b
                buf, sem = bufs[b], sems[b]
                pltpu.make_async_copy(tab_hbm.at[idx_v.at[jj]], buf, sem).wait()
                _scale_buf(buf)
                pltpu.sync_copy(buf, out_hbm.at[pl.ds(base + jj * K, K)])

                @pl.when(jj + 2 < CH)
                def _next():
                    pltpu.async_copy(tab_hbm.at[idx_v.at[jj + 2]], buf, sem)

    return emb


_emb = _make_sc_kernel()


def kernel(x, table):
    xf = x.astype(jnp.int32).reshape(NW * CH, K)
    out = _emb(xf, table)
    return out.reshape(x.shape[0], x.shape[1], D)
